# initial kernel scaffold (unmeasured)
import jax
import jax.numpy as jnp
from jax import lax
from jax.experimental import pallas as pl
from jax.experimental.pallas import tpu as pltpu

N_DEV = 4
M_PER = 2048
K = 8192
N_PER = 1024
M_TILE = 256


def _ag_body(x_ref, xf_ref, copy_sem, send_sems, recv_sems):
    my = lax.axis_index("i")
    left = (my - 1) % N_DEV
    right = (my + 1) % N_DEV

    barrier_sem = pltpu.get_barrier_semaphore()
    for nbr in [left, right]:
        pl.semaphore_signal(
            barrier_sem, inc=1,
            device_id=(nbr,), device_id_type=pl.DeviceIdType.MESH,
        )
    pl.semaphore_wait(barrier_sem, 2)

    cp = pltpu.make_async_copy(x_ref, xf_ref.at[my], copy_sem)
    cp.start()
    cp.wait()

    for h in range(N_DEV - 1):
        origin = (my - h) % N_DEV
        rdma = pltpu.make_async_remote_copy(
            src_ref=xf_ref.at[origin],
            dst_ref=xf_ref.at[origin],
            send_sem=send_sems.at[h],
            recv_sem=recv_sems.at[h],
            device_id=(right,),
            device_id_type=pl.DeviceIdType.MESH,
        )
        rdma.start()
        rdma.wait()


def _all_gather_x(x_shard):
    return pl.pallas_call(
        _ag_body,
        out_shape=jax.ShapeDtypeStruct((N_DEV, M_PER, K), x_shard.dtype),
        in_specs=[pl.BlockSpec(memory_space=pltpu.ANY)],
        out_specs=pl.BlockSpec(memory_space=pltpu.ANY),
        scratch_shapes=[
            pltpu.SemaphoreType.DMA,
            pltpu.SemaphoreType.DMA((N_DEV - 1,)),
            pltpu.SemaphoreType.DMA((N_DEV - 1,)),
        ],
        compiler_params=pltpu.CompilerParams(collective_id=0),
    )(x_shard)


def _gemm_body(x_blk, w_ref, o_ref):
    y = jnp.dot(x_blk[0], w_ref[:], preferred_element_type=jnp.float32)
    o_ref[:] = y * jax.nn.sigmoid(y)


def _gemm_silu(x_full, w_shard):
    tiles_per_chunk = M_PER // M_TILE
    grid = (N_DEV, tiles_per_chunk)
    return pl.pallas_call(
        _gemm_body,
        grid=grid,
        out_shape=jax.ShapeDtypeStruct((N_DEV * M_PER, N_PER), jnp.float32),
        in_specs=[
            pl.BlockSpec((1, M_TILE, K), lambda c, i: (c, i, 0)),
            pl.BlockSpec((K, N_PER), lambda c, i: (0, 0)),
        ],
        out_specs=pl.BlockSpec(
            (M_TILE, N_PER), lambda c, i: (c * (M_PER // M_TILE) + i, 0)
        ),
    )(x_full, w_shard)


def kernel(x, w_mat):
    x_full = _all_gather_x(x)
    return _gemm_silu(x_full, w_mat)


# baseline (device time: 4375588 ns/iter reference)
import jax
import jax.numpy as jnp
from jax import lax
from jax.experimental import pallas as pl
from jax.experimental.pallas import tpu as pltpu

N_DEV = 4
M_PER = 2048
K = 8192
N_PER = 1024
M_TILE = 256


def _ag_body(x_ref, xf_ref, copy_sem, send_sems, recv_sems):
    my = lax.axis_index("i")
    left = (my - 1) % N_DEV
    right = (my + 1) % N_DEV

    barrier_sem = pltpu.get_barrier_semaphore()
    for nbr in [left, right]:
        pl.semaphore_signal(
            barrier_sem, inc=1,
            device_id=(nbr,), device_id_type=pl.DeviceIdType.MESH,
        )
    pl.semaphore_wait(barrier_sem, 2)

    cp = pltpu.make_async_copy(x_ref, xf_ref.at[my], copy_sem)
    cp.start()
    cp.wait()

    for h in range(N_DEV - 1):
        origin = (my - h) % N_DEV
        rdma = pltpu.make_async_remote_copy(
            src_ref=xf_ref.at[origin],
            dst_ref=xf_ref.at[origin],
            send_sem=send_sems.at[h],
            recv_sem=recv_sems.at[h],
            device_id=(right,),
            device_id_type=pl.DeviceIdType.MESH,
        )
        rdma.start()
        rdma.wait()


def _all_gather_x(x_shard):
    return pl.pallas_call(
        _ag_body,
        out_shape=jax.ShapeDtypeStruct((N_DEV, M_PER, K), x_shard.dtype),
        in_specs=[pl.BlockSpec(memory_space=pl.ANY)],
        out_specs=pl.BlockSpec(memory_space=pl.ANY),
        scratch_shapes=[
            pltpu.SemaphoreType.DMA,
            pltpu.SemaphoreType.DMA((N_DEV - 1,)),
            pltpu.SemaphoreType.DMA((N_DEV - 1,)),
        ],
        compiler_params=pltpu.CompilerParams(collective_id=0),
    )(x_shard)


def _gemm_body(x_blk, w_ref, o_ref):
    y = jnp.dot(x_blk[0], w_ref[:], preferred_element_type=jnp.float32)
    o_ref[:] = y * jax.nn.sigmoid(y)


def _gemm_silu(x_full, w_shard):
    tiles_per_chunk = M_PER // M_TILE
    grid = (N_DEV, tiles_per_chunk)
    return pl.pallas_call(
        _gemm_body,
        grid=grid,
        out_shape=jax.ShapeDtypeStruct((N_DEV * M_PER, N_PER), jnp.float32),
        in_specs=[
            pl.BlockSpec((1, M_TILE, K), lambda c, i: (c, i, 0)),
            pl.BlockSpec((K, N_PER), lambda c, i: (0, 0)),
        ],
        out_specs=pl.BlockSpec(
            (M_TILE, N_PER), lambda c, i: (c * (M_PER // M_TILE) + i, 0)
        ),
        compiler_params=pltpu.CompilerParams(
            vmem_limit_bytes=60 * 1024 * 1024,
        ),
    )(x_full, w_shard)


def kernel(x, w_mat):
    x_full = _all_gather_x(x)
    return _gemm_silu(x_full, w_mat)


# device time: 2322206 ns/iter; 1.8842x vs baseline; 1.8842x over previous
import jax
import jax.numpy as jnp
from jax import lax
from jax.experimental import pallas as pl
from jax.experimental.pallas import tpu as pltpu

N_DEV = 4
M_PER = 2048
K = 8192
N_PER = 1024
M_TILE = 256


def _ag_body(x_ref, xf_ref, copy_sem, send_sems, recv_sems):
    my = lax.axis_index("i")
    left = (my - 1) % N_DEV
    right = (my + 1) % N_DEV

    barrier_sem = pltpu.get_barrier_semaphore()
    for nbr in [left, right]:
        pl.semaphore_signal(
            barrier_sem, inc=1,
            device_id=(nbr,), device_id_type=pl.DeviceIdType.MESH,
        )
    pl.semaphore_wait(barrier_sem, 2)

    cp = pltpu.make_async_copy(x_ref, xf_ref.at[my], copy_sem)
    cp.start()
    cp.wait()

    for h in range(N_DEV - 1):
        origin = (my - h) % N_DEV
        rdma = pltpu.make_async_remote_copy(
            src_ref=xf_ref.at[origin],
            dst_ref=xf_ref.at[origin],
            send_sem=send_sems.at[h],
            recv_sem=recv_sems.at[h],
            device_id=(right,),
            device_id_type=pl.DeviceIdType.MESH,
        )
        rdma.start()
        rdma.wait()


def _all_gather_x(x_shard):
    return pl.pallas_call(
        _ag_body,
        out_shape=jax.ShapeDtypeStruct((N_DEV, M_PER, K), x_shard.dtype),
        in_specs=[pl.BlockSpec(memory_space=pl.ANY)],
        out_specs=pl.BlockSpec(memory_space=pl.ANY),
        scratch_shapes=[
            pltpu.SemaphoreType.DMA,
            pltpu.SemaphoreType.DMA((N_DEV - 1,)),
            pltpu.SemaphoreType.DMA((N_DEV - 1,)),
        ],
        compiler_params=pltpu.CompilerParams(collective_id=0),
    )(x_shard)


def _gemm_body(x_blk, w_ref, o_ref):
    y = jnp.dot(x_blk[0], w_ref[:], preferred_element_type=jnp.float32)
    o_ref[:] = y * jax.nn.sigmoid(y)


def _gemm_silu(x_full, w_shard):
    tiles_per_chunk = M_PER // M_TILE
    grid = (N_DEV, tiles_per_chunk)
    return pl.pallas_call(
        _gemm_body,
        grid=grid,
        out_shape=jax.ShapeDtypeStruct((N_DEV * M_PER, N_PER), jnp.float32),
        in_specs=[
            pl.BlockSpec((1, M_TILE, K), lambda c, i: (c, i, 0)),
            pl.BlockSpec((K, N_PER), lambda c, i: (0, 0)),
        ],
        out_specs=pl.BlockSpec(
            (M_TILE, N_PER), lambda c, i: (c * (M_PER // M_TILE) + i, 0)
        ),
        compiler_params=pltpu.CompilerParams(
            vmem_limit_bytes=60 * 1024 * 1024,
        ),
    )(x_full, w_shard)


def kernel(x, w_mat):
    x_full = _all_gather_x(x.astype(jnp.bfloat16))
    return _gemm_silu(x_full, w_mat.astype(jnp.bfloat16))
